# baseline (speedup 1.0000x reference)
import functools

import jax
import jax.numpy as jnp
from jax import lax
from jax.experimental import pallas as pl
from jax.experimental.pallas import tpu as pltpu
from jax.experimental.pallas import tpu_sc as plsc

_B = 16384
_V = 1000000
_D = 64
_H = 128

_NC = 2
_NS = 16
_NW = _NC * _NS
_BPW = _B // _NW
_L = 16
_CH = 256
_NCHUNK = _BPW // _CH


@functools.cache
def _make_sc_gather2():
    mesh = plsc.VectorSubcoreMesh(core_axis_name="c", subcore_axis_name="s")

    @functools.partial(
        pl.kernel,
        mesh=mesh,
        out_type=jax.ShapeDtypeStruct((_B, _D), jnp.float32),
        scratch_types=[
            pltpu.VMEM((_BPW,), jnp.int32),
            pltpu.VMEM((_BPW,), jnp.int32),
            pltpu.VMEM((_CH, _D), jnp.float32),
            pltpu.VMEM((_CH, _D), jnp.float32),
            pltpu.SemaphoreType.DMA,
        ],
    )
    def _sc_gather2(t1_hbm, t2_hbm, i0_hbm, i1_hbm, o_hbm,
                    raw0_v, raw1_v, r1_v, r2_v, sem):
        wid = lax.axis_index("s") * _NC + lax.axis_index("c")
        base = wid * _BPW
        pltpu.sync_copy(i0_hbm.at[wid], raw0_v)
        pltpu.sync_copy(i1_hbm.at[wid], raw1_v)

        for ch in range(_NCHUNK):
            ibase = ch * _CH

            def issue(g):
                g16 = g * _L
                v0 = raw0_v[pl.ds(ibase + g16, _L)]
                v1 = raw1_v[pl.ds(ibase + g16, _L)]
                for l in range(_L):
                    pltpu.async_copy(t1_hbm.at[v0[l]], r1_v.at[g16 + l], sem)
                    pltpu.async_copy(t2_hbm.at[v1[l]], r2_v.at[g16 + l], sem)

            pl.loop(0, _CH // _L)(issue)

            def drain(i):
                pltpu.make_async_copy(t1_hbm.at[0], r1_v.at[i], sem).wait()
                pltpu.make_async_copy(t2_hbm.at[0], r2_v.at[i], sem).wait()

            pl.loop(0, _CH)(drain)

            def accum(i):
                for j in range(_D // _L):
                    sl = pl.ds(j * _L, _L)
                    r1_v[i, sl] = r1_v[i, sl] + r2_v[i, sl]

            pl.loop(0, _CH)(accum)
            pltpu.sync_copy(r1_v, o_hbm.at[pl.ds(base + ibase, _CH)])

    return _sc_gather2


def _tc_mlp_body(e_ref, W1_ref, b1_ref, s1_ref, t1_ref,
                 W2_ref, b2_ref, s2_ref, t2_ref, o_ref):
    h = jnp.dot(e_ref[...], W1_ref[...], preferred_element_type=jnp.float32)
    h = h + b1_ref[...]
    h = jnp.where(h > 0, h, 0.01 * h)
    h = h * s1_ref[...] + t1_ref[...]
    h = jnp.dot(h, W2_ref[...], preferred_element_type=jnp.float32) + b2_ref[...]
    h = jnp.where(h > 0, h, 0.01 * h)
    o_ref[...] = h * s2_ref[...] + t2_ref[...]


_BM = 2048


def _tc_mlp(e, W1, b1, s1, t1, W2, b2, s2, t2):
    grid = (_B // _BM,)
    full = lambda shape: pl.BlockSpec(shape, lambda i: (0, 0))
    return pl.pallas_call(
        _tc_mlp_body,
        grid=grid,
        in_specs=[
            pl.BlockSpec((_BM, _D), lambda i: (i, 0)),
            full((_D, 2 * _D)),
            full((1, 2 * _D)),
            full((1, 2 * _D)),
            full((1, 2 * _D)),
            full((2 * _D, _H)),
            full((1, _H)),
            full((1, _H)),
            full((1, _H)),
        ],
        out_specs=pl.BlockSpec((_BM, _H), lambda i: (i, 0)),
        out_shape=jax.ShapeDtypeStruct((_B, _H), jnp.float32),
    )(e, W1, b1, s1, t1, W2, b2, s2, t2)


def kernel(input_features, bb1_table, reaction_table, W1, b1, g1, be1, rm1,
           rv1, W2, b2, g2, be2, rm2, rv2):
    idx = input_features.reshape(_NW, _BPW, 2)
    idx0 = idx[..., 0]
    idx1 = idx[..., 1]
    e = _make_sc_gather2()(bb1_table, reaction_table, idx0, idx1)
    s1 = g1 * lax.rsqrt(rv1 + 1e-5)
    t1v = be1 - rm1 * s1
    s2 = g2 * lax.rsqrt(rv2 + 1e-5)
    t2v = be2 - rm2 * s2
    r = lambda v: v.reshape(1, -1)
    return _tc_mlp(e, W1, r(b1), r(s1), r(t1v), W2, r(b2), r(s2), r(t2v))


# speedup vs baseline: 1.4013x; 1.4013x over previous
import functools

import jax
import jax.numpy as jnp
from jax import lax
from jax.experimental import pallas as pl
from jax.experimental.pallas import tpu as pltpu
from jax.experimental.pallas import tpu_sc as plsc

_B = 16384
_V = 1000000
_D = 64
_H = 128

_NC = 2
_NS = 16
_NW = _NC * _NS
_BPW = _B // _NW
_L = 16
_CH = 256
_NCHUNK = _BPW // _CH


@functools.cache
def _make_sc_gather2():
    mesh = plsc.VectorSubcoreMesh(core_axis_name="c", subcore_axis_name="s")

    @functools.partial(
        pl.kernel,
        mesh=mesh,
        out_type=jax.ShapeDtypeStruct((_B, _D), jnp.float32),
        scratch_types=[
            pltpu.VMEM((_BPW,), jnp.int32),
            pltpu.VMEM((_BPW,), jnp.int32),
            pltpu.VMEM((_CH, _D), jnp.float32),
            pltpu.VMEM((_CH, _D), jnp.float32),
            pltpu.SemaphoreType.DMA,
        ],
    )
    def _sc_gather2(t1_hbm, t2_hbm, i0_hbm, i1_hbm, o_hbm,
                    raw0_v, raw1_v, r1_v, r2_v, sem):
        wid = lax.axis_index("s") * _NC + lax.axis_index("c")
        base = wid * _BPW
        pltpu.sync_copy(i0_hbm.at[wid], raw0_v)
        pltpu.sync_copy(i1_hbm.at[wid], raw1_v)

        for ch in range(_NCHUNK):
            ibase = ch * _CH

            def issue(g):
                g16 = g * _L
                v0 = raw0_v[pl.ds(ibase + g16, _L)]
                v1 = raw1_v[pl.ds(ibase + g16, _L)]
                for l in range(_L):
                    pltpu.async_copy(t1_hbm.at[v0[l]], r1_v.at[g16 + l], sem)
                    pltpu.async_copy(t2_hbm.at[v1[l] >> 3, v1[l] & 7],
                                     r2_v.at[g16 + l], sem)

            pl.loop(0, _CH // _L)(issue)

            def drain(i):
                pltpu.make_async_copy(t1_hbm.at[0], r1_v.at[i], sem).wait()
                pltpu.make_async_copy(t2_hbm.at[0, 0], r2_v.at[i], sem).wait()

            pl.loop(0, _CH)(drain)

            def accum(i):
                for j in range(_D // _L):
                    sl = pl.ds(j * _L, _L)
                    r1_v[i, sl] = r1_v[i, sl] + r2_v[i, sl]

            pl.loop(0, _CH)(accum)
            pltpu.sync_copy(r1_v, o_hbm.at[pl.ds(base + ibase, _CH)])

    return _sc_gather2


def _tc_mlp_body(e_ref, W1_ref, b1_ref, s1_ref, t1_ref,
                 W2_ref, b2_ref, s2_ref, t2_ref, o_ref):
    h = jnp.dot(e_ref[...], W1_ref[...], preferred_element_type=jnp.float32)
    h = h + b1_ref[...]
    h = jnp.where(h > 0, h, 0.01 * h)
    h = h * s1_ref[...] + t1_ref[...]
    h = jnp.dot(h, W2_ref[...], preferred_element_type=jnp.float32) + b2_ref[...]
    h = jnp.where(h > 0, h, 0.01 * h)
    o_ref[...] = h * s2_ref[...] + t2_ref[...]


_BM = 2048


def _tc_mlp(e, W1, b1, s1, t1, W2, b2, s2, t2):
    grid = (_B // _BM,)
    full = lambda shape: pl.BlockSpec(shape, lambda i: (0, 0))
    return pl.pallas_call(
        _tc_mlp_body,
        grid=grid,
        in_specs=[
            pl.BlockSpec((_BM, _D), lambda i: (i, 0)),
            full((_D, 2 * _D)),
            full((1, 2 * _D)),
            full((1, 2 * _D)),
            full((1, 2 * _D)),
            full((2 * _D, _H)),
            full((1, _H)),
            full((1, _H)),
            full((1, _H)),
        ],
        out_specs=pl.BlockSpec((_BM, _H), lambda i: (i, 0)),
        out_shape=jax.ShapeDtypeStruct((_B, _H), jnp.float32),
    )(e, W1, b1, s1, t1, W2, b2, s2, t2)


def kernel(input_features, bb1_table, reaction_table, W1, b1, g1, be1, rm1,
           rv1, W2, b2, g2, be2, rm2, rv2):
    t2 = reaction_table.reshape(_V // 8, 8, _D)
    idx = input_features.reshape(_NW, _BPW, 2)
    idx0 = idx[..., 0]
    idx1 = idx[..., 1]
    e = _make_sc_gather2()(bb1_table, t2, idx0, idx1)
    s1 = g1 * lax.rsqrt(rv1 + 1e-5)
    t1v = be1 - rm1 * s1
    s2 = g2 * lax.rsqrt(rv2 + 1e-5)
    t2v = be2 - rm2 * s2
    r = lambda v: v.reshape(1, -1)
    return _tc_mlp(e, W1, r(b1), r(s1), r(t1v), W2, r(b2), r(s2), r(t2v))


# speedup vs baseline: 1.7049x; 1.2166x over previous
import functools

import jax
import jax.numpy as jnp
from jax import lax
from jax.experimental import pallas as pl
from jax.experimental.pallas import tpu as pltpu
from jax.experimental.pallas import tpu_sc as plsc

_B = 16384
_V = 1000000
_D = 64
_H = 128

_NC = 2
_NS = 16
_NW = _NC * _NS
_L = 16
_NB = (_V + 127) // 128
_BASE_BLKS = _NB // _NW
_EXTRA = _NB % _NW
_LAST = _NB - 1
_ROWBUF = 256
_FLUSH_AT = _ROWBUF - _L
_SRT = 20480


@functools.cache
def _make_sc_gather2():
    mesh = plsc.VectorSubcoreMesh(core_axis_name="c", subcore_axis_name="s")

    @functools.partial(
        pl.kernel,
        mesh=mesh,
        out_type=[
            jax.ShapeDtypeStruct((_B, _D), jnp.float32),
            jax.ShapeDtypeStruct((_B, _D), jnp.float32),
        ],
        scratch_types=[
            pltpu.VMEM((2048,), jnp.int32),
            pltpu.VMEM((_B + _L,), jnp.int32),
            pltpu.VMEM((_SRT,), jnp.int32),
            pltpu.VMEM((256,), jnp.int32),
            pltpu.VMEM((2, _D, 128), jnp.float32),
            pltpu.VMEM((_ROWBUF, _D), jnp.float32),
            pltpu.VMEM((_ROWBUF,), jnp.int32),
            pltpu.SMEM((257,), jnp.int32),
            pltpu.SMEM((257,), jnp.int32),
            pltpu.SMEM((257,), jnp.int32),
            pltpu.SMEM((1,), jnp.int32),
            pltpu.SemaphoreType.DMA,
            pltpu.SemaphoreType.DMA,
        ],
        compiler_params=pltpu.CompilerParams(needs_layout_passes=False),
    )
    def _sc_gather2(t1_hbm, t2_hbm, i0_hbm, i1_hbm, o1_hbm, o2_hbm,
                    idx_v, uns_v, srt_v, cnt_v, blk_v, row_v, pos_v,
                    start_s, cur_s, ncnt_s, rows_s, semf, semo):
        wid = lax.axis_index("s") * _NC + lax.axis_index("c")
        cstart = wid * _BASE_BLKS + jnp.minimum(wid, _EXTRA)
        count = _BASE_BLKS + jnp.where(wid < _EXTRA, 1, 0)
        cend = cstart + count
        lanes = lax.iota(jnp.int32, _L)
        m0 = lanes == 0
        zeros16 = jnp.zeros((_L,), jnp.int32)
        ones16 = jnp.ones((_L,), jnp.int32)

        for t in range(2):
            tT = (t1_hbm, t2_hbm)[t]
            isrc = (i0_hbm, i1_hbm)[t]
            o = (o1_hbm, o2_hbm)[t]

            for q in range(256 // _L):
                cnt_v[pl.ds(q * _L, _L)] = zeros16

            nhits = 0
            for chk in range(_B // 2048):
                pltpu.sync_copy(isrc.at[pl.ds(chk * 2048, 2048)], idx_v)

                def scan(g, off, chk=chk):
                    v = idx_v[pl.ds(g * _L, _L)]
                    c = v >> 7
                    m = (c >= cstart) & (c < cend)
                    relc = jnp.where(m, c - cstart, 0)
                    plsc.addupdate_scatter(cnt_v, [relc], ones16, mask=m)
                    packed = ((chk * 2048 + g * _L + lanes)
                              | ((v & 127) << 14) | (relc << 21))
                    plsc.store_compressed(uns_v.at[pl.ds(off, _L)], packed,
                                          mask=m)
                    n = plsc.all_reduce_population_count(m)
                    return off + n[0]

                nhits = pl.loop(0, 2048 // _L, init_carry=nhits)(scan)

            base = 0
            for q in range(256 // _L):
                cv = cnt_v[pl.ds(q * _L, _L)]
                for l in range(_L):
                    k = q * _L + l
                    start_s[k] = base
                    cur_s[k] = base
                    ncnt_s[k] = cv[l]
                    base = base + ((cv[l] + _L - 1) & -_L)

            def place(g):
                v = uns_v[pl.ds(g * _L, _L)]
                for l in range(_L):
                    @pl.when(g * _L + l < nhits)
                    def _():
                        p = v[l]
                        relc = p >> 21
                        slot = cur_s[relc]
                        cur_s[relc] = slot + 1
                        plsc.store_scatter(
                            srt_v, [jnp.full((_L,), slot, jnp.int32)],
                            jnp.full((_L,), p, jnp.int32), mask=m0)

            pl.loop(0, (nhits + _L - 1) // _L)(place)

            rows_s[0] = 0

            def flush():
                cnt = rows_s[0]

                def fire(g):
                    pv = pos_v[pl.ds(g * _L, _L)]
                    for l in range(_L):
                        @pl.when(g * _L + l < cnt)
                        def _():
                            pltpu.async_copy(row_v.at[g * _L + l],
                                             o.at[pv[l]], semo)

                pl.loop(0, (cnt + _L - 1) // _L)(fire)

                def drain(i):
                    pltpu.make_async_copy(o.at[0], row_v.at[0], semo).wait()

                pl.loop(0, cnt)(drain)
                rows_s[0] = 0

            def fetch(j):
                @pl.when(j < count)
                def _():
                    off = pl.multiple_of((cstart + j) * 128, 128)
                    pltpu.async_copy(tT.at[:, pl.ds(off, 128)],
                                     blk_v.at[j & 1], semf)

            def wait_fetch(j):
                pltpu.make_async_copy(tT.at[:, pl.ds(0, 128)],
                                      blk_v.at[j & 1], semf).wait()

            fetch(0)

            def block_body(k):
                fetch(k + 1)
                wait_fetch(k)
                pblk = k & 1
                st = start_s[k]
                n = ncnt_s[k]

                def group(g):
                    @pl.when(rows_s[0] >= _FLUSH_AT)
                    def _():
                        flush()

                    v = srt_v[pl.ds(st + g * _L, _L)]
                    for l in range(_L):
                        @pl.when(g * _L + l < n)
                        def _():
                            p = v[l]
                            pos = p & (_B - 1)
                            lane = (p >> 14) & 127
                            rc = rows_s[0]
                            rows_s[0] = rc + 1
                            plsc.store_scatter(
                                pos_v, [jnp.full((_L,), rc, jnp.int32)],
                                jnp.full((_L,), pos, jnp.int32), mask=m0)
                            lv = jnp.full((_L,), lane, jnp.int32)
                            pv = jnp.full((_L,), pblk, jnp.int32)
                            for q in range(_D // _L):
                                g_ = plsc.load_gather(
                                    blk_v, [pv, lanes + q * _L, lv])
                                row_v[rc, pl.ds(q * _L, _L)] = g_

                pl.loop(0, (n + _L - 1) // _L)(group)

            pl.loop(0, count)(block_body)
            flush()

    return _sc_gather2


def _tc_mlp_body(e1_ref, e2_ref, W1_ref, b1_ref, s1_ref, t1_ref,
                 W2_ref, b2_ref, s2_ref, t2_ref, o_ref):
    x = e1_ref[...] + e2_ref[...]
    h = jnp.dot(x, W1_ref[...], preferred_element_type=jnp.float32)
    h = h + b1_ref[...]
    h = jnp.where(h > 0, h, 0.01 * h)
    h = h * s1_ref[...] + t1_ref[...]
    h = jnp.dot(h, W2_ref[...], preferred_element_type=jnp.float32) + b2_ref[...]
    h = jnp.where(h > 0, h, 0.01 * h)
    o_ref[...] = h * s2_ref[...] + t2_ref[...]


_BM = 2048


def _tc_mlp(e1, e2, W1, b1, s1, t1, W2, b2, s2, t2):
    grid = (_B // _BM,)
    full = lambda shape: pl.BlockSpec(shape, lambda i: (0, 0))
    return pl.pallas_call(
        _tc_mlp_body,
        grid=grid,
        in_specs=[
            pl.BlockSpec((_BM, _D), lambda i: (i, 0)),
            pl.BlockSpec((_BM, _D), lambda i: (i, 0)),
            full((_D, 2 * _D)),
            full((1, 2 * _D)),
            full((1, 2 * _D)),
            full((1, 2 * _D)),
            full((2 * _D, _H)),
            full((1, _H)),
            full((1, _H)),
            full((1, _H)),
        ],
        out_specs=pl.BlockSpec((_BM, _H), lambda i: (i, 0)),
        out_shape=jax.ShapeDtypeStruct((_B, _H), jnp.float32),
    )(e1, e2, W1, b1, s1, t1, W2, b2, s2, t2)


def kernel(input_features, bb1_table, reaction_table, W1, b1, g1, be1, rm1,
           rv1, W2, b2, g2, be2, rm2, rv2):
    t1 = bb1_table.T
    t2 = reaction_table.T
    idx0 = input_features[:, 0]
    idx1 = input_features[:, 1]
    e1, e2 = _make_sc_gather2()(t1, t2, idx0, idx1)
    s1 = g1 * lax.rsqrt(rv1 + 1e-5)
    t1v = be1 - rm1 * s1
    s2 = g2 * lax.rsqrt(rv2 + 1e-5)
    t2v = be2 - rm2 * s2
    r = lambda v: v.reshape(1, -1)
    return _tc_mlp(e1, e2, W1, r(b1), r(s1), r(t1v), W2, r(b2), r(s2), r(t2v))


# speedup vs baseline: 2.2514x; 1.3206x over previous
import functools

import jax
import jax.numpy as jnp
from jax import lax
from jax.experimental import pallas as pl
from jax.experimental.pallas import tpu as pltpu
from jax.experimental.pallas import tpu_sc as plsc

_B = 16384
_V = 1000000
_D = 64
_H = 128

_NC = 2
_NS = 16
_NW = _NC * _NS
_L = 16
_NB = (_V + 255) // 256
_BASE_BLKS = _NB // _NW
_EXTRA = _NB % _NW
_LAST = _NB - 1
_ROWBUF = 256
_FLUSH_AT = _ROWBUF - _L
_SRT = 20480


@functools.cache
def _make_sc_gather2():
    mesh = plsc.VectorSubcoreMesh(core_axis_name="c", subcore_axis_name="s")

    @functools.partial(
        pl.kernel,
        mesh=mesh,
        out_type=[
            jax.ShapeDtypeStruct((_B, _D), jnp.float32),
            jax.ShapeDtypeStruct((_B, _D), jnp.float32),
        ],
        scratch_types=[
            pltpu.VMEM((2048,), jnp.int32),
            pltpu.VMEM((_B + _L,), jnp.int32),
            pltpu.VMEM((_SRT,), jnp.int32),
            pltpu.VMEM((256,), jnp.int32),
            pltpu.VMEM((2, _D, 256), jnp.float32),
            pltpu.VMEM((_ROWBUF, _D), jnp.float32),
            pltpu.VMEM((_ROWBUF,), jnp.int32),
            pltpu.SMEM((257,), jnp.int32),
            pltpu.SMEM((257,), jnp.int32),
            pltpu.SMEM((257,), jnp.int32),
            pltpu.SMEM((1,), jnp.int32),
            pltpu.SemaphoreType.DMA,
            pltpu.SemaphoreType.DMA,
        ],
        compiler_params=pltpu.CompilerParams(needs_layout_passes=False),
    )
    def _sc_gather2(t1_hbm, t2_hbm, i0_hbm, i1_hbm, o1_hbm, o2_hbm,
                    idx_v, uns_v, srt_v, cnt_v, blk_v, row_v, pos_v,
                    start_s, cur_s, ncnt_s, rows_s, semf, semo):
        wid = lax.axis_index("s") * _NC + lax.axis_index("c")
        cstart = wid * _BASE_BLKS + jnp.minimum(wid, _EXTRA)
        count = _BASE_BLKS + jnp.where(wid < _EXTRA, 1, 0)
        cend = cstart + count
        lanes = lax.iota(jnp.int32, _L)
        m0 = lanes == 0
        zeros16 = jnp.zeros((_L,), jnp.int32)
        ones16 = jnp.ones((_L,), jnp.int32)

        for t in range(2):
            tT = (t1_hbm, t2_hbm)[t]
            isrc = (i0_hbm, i1_hbm)[t]
            o = (o1_hbm, o2_hbm)[t]

            for q in range(256 // _L):
                cnt_v[pl.ds(q * _L, _L)] = zeros16

            nhits = 0
            for chk in range(_B // 2048):
                pltpu.sync_copy(isrc.at[pl.ds(chk * 2048, 2048)], idx_v)

                def scan(g, off, chk=chk):
                    v = idx_v[pl.ds(g * _L, _L)]
                    c = v >> 8
                    m = (c >= cstart) & (c < cend)
                    relc = jnp.where(m, c - cstart, 0)
                    plsc.addupdate_scatter(cnt_v, [relc], ones16, mask=m)
                    packed = ((chk * 2048 + g * _L + lanes)
                              | ((v & 255) << 14) | (relc << 22))
                    plsc.store_compressed(uns_v.at[pl.ds(off, _L)], packed,
                                          mask=m)
                    n = plsc.all_reduce_population_count(m)
                    return off + n[0]

                nhits = pl.loop(0, 2048 // _L, init_carry=nhits,
                                unroll=4)(scan)

            base = 0
            for q in range(256 // _L):
                cv = cnt_v[pl.ds(q * _L, _L)]
                for l in range(_L):
                    k = q * _L + l
                    start_s[k] = base
                    cur_s[k] = base
                    ncnt_s[k] = cv[l]
                    base = base + ((cv[l] + _L - 1) & -_L)

            def place(g):
                v = uns_v[pl.ds(g * _L, _L)]
                for l in range(_L):
                    @pl.when(g * _L + l < nhits)
                    def _():
                        p = v[l]
                        relc = p >> 22
                        slot = cur_s[relc]
                        cur_s[relc] = slot + 1
                        plsc.store_scatter(
                            srt_v, [jnp.full((_L,), slot, jnp.int32)],
                            jnp.full((_L,), p, jnp.int32), mask=m0)

            pl.loop(0, (nhits + _L - 1) // _L)(place)

            rows_s[0] = 0

            def flush():
                cnt = rows_s[0]

                def fire(g):
                    pv = pos_v[pl.ds(g * _L, _L)]
                    for l in range(_L):
                        @pl.when(g * _L + l < cnt)
                        def _():
                            pltpu.async_copy(row_v.at[g * _L + l],
                                             o.at[pv[l]], semo)

                pl.loop(0, (cnt + _L - 1) // _L)(fire)

                def drain(i):
                    pltpu.make_async_copy(o.at[0], row_v.at[0], semo).wait()

                pl.loop(0, cnt)(drain)
                rows_s[0] = 0

            def fetch(j):
                @pl.when(j < count)
                def _():
                    cc = cstart + j
                    off = pl.multiple_of(cc * 256, 128)
                    p = j & 1

                    @pl.when(cc == _LAST)
                    def _():
                        pltpu.async_copy(tT.at[:, pl.ds(off, 128)],
                                         blk_v.at[p, :, pl.ds(0, 128)], semf)

                    @pl.when(cc != _LAST)
                    def _():
                        pltpu.async_copy(tT.at[:, pl.ds(off, 256)],
                                         blk_v.at[p], semf)

            def wait_fetch(j):
                cc = cstart + j
                p = j & 1

                @pl.when(cc == _LAST)
                def _():
                    pltpu.make_async_copy(tT.at[:, pl.ds(0, 128)],
                                          blk_v.at[p, :, pl.ds(0, 128)],
                                          semf).wait()

                @pl.when(cc != _LAST)
                def _():
                    pltpu.make_async_copy(tT.at[:, pl.ds(0, 256)],
                                          blk_v.at[p], semf).wait()

            fetch(0)

            def block_body(k):
                fetch(k + 1)
                wait_fetch(k)
                pblk = k & 1
                st = start_s[k]
                n = ncnt_s[k]

                def group(g):
                    @pl.when(rows_s[0] >= _FLUSH_AT)
                    def _():
                        flush()

                    v = srt_v[pl.ds(st + g * _L, _L)]
                    for l in range(_L):
                        @pl.when(g * _L + l < n)
                        def _():
                            p = v[l]
                            pos = p & (_B - 1)
                            lane = (p >> 14) & 255
                            rc = rows_s[0]
                            rows_s[0] = rc + 1
                            plsc.store_scatter(
                                pos_v, [jnp.full((_L,), rc, jnp.int32)],
                                jnp.full((_L,), pos, jnp.int32), mask=m0)
                            lv = jnp.full((_L,), lane, jnp.int32)
                            pv = jnp.full((_L,), pblk, jnp.int32)
                            for q in range(_D // _L):
                                g_ = plsc.load_gather(
                                    blk_v, [pv, lanes + q * _L, lv])
                                row_v[rc, pl.ds(q * _L, _L)] = g_

                pl.loop(0, (n + _L - 1) // _L)(group)

            pl.loop(0, count)(block_body)
            flush()

    return _sc_gather2


def _tc_mlp_body(e1_ref, e2_ref, W1_ref, b1_ref, s1_ref, t1_ref,
                 W2_ref, b2_ref, s2_ref, t2_ref, o_ref):
    x = e1_ref[...] + e2_ref[...]
    h = jnp.dot(x, W1_ref[...], preferred_element_type=jnp.float32)
    h = h + b1_ref[...]
    h = jnp.where(h > 0, h, 0.01 * h)
    h = h * s1_ref[...] + t1_ref[...]
    h = jnp.dot(h, W2_ref[...], preferred_element_type=jnp.float32) + b2_ref[...]
    h = jnp.where(h > 0, h, 0.01 * h)
    o_ref[...] = h * s2_ref[...] + t2_ref[...]


_BM = 2048


def _tc_mlp(e1, e2, W1, b1, s1, t1, W2, b2, s2, t2):
    grid = (_B // _BM,)
    full = lambda shape: pl.BlockSpec(shape, lambda i: (0, 0))
    return pl.pallas_call(
        _tc_mlp_body,
        grid=grid,
        in_specs=[
            pl.BlockSpec((_BM, _D), lambda i: (i, 0)),
            pl.BlockSpec((_BM, _D), lambda i: (i, 0)),
            full((_D, 2 * _D)),
            full((1, 2 * _D)),
            full((1, 2 * _D)),
            full((1, 2 * _D)),
            full((2 * _D, _H)),
            full((1, _H)),
            full((1, _H)),
            full((1, _H)),
        ],
        out_specs=pl.BlockSpec((_BM, _H), lambda i: (i, 0)),
        out_shape=jax.ShapeDtypeStruct((_B, _H), jnp.float32),
    )(e1, e2, W1, b1, s1, t1, W2, b2, s2, t2)


def kernel(input_features, bb1_table, reaction_table, W1, b1, g1, be1, rm1,
           rv1, W2, b2, g2, be2, rm2, rv2):
    t1 = bb1_table.T
    t2 = reaction_table.T
    idx0 = input_features[:, 0]
    idx1 = input_features[:, 1]
    e1, e2 = _make_sc_gather2()(t1, t2, idx0, idx1)
    s1 = g1 * lax.rsqrt(rv1 + 1e-5)
    t1v = be1 - rm1 * s1
    s2 = g2 * lax.rsqrt(rv2 + 1e-5)
    t2v = be2 - rm2 * s2
    r = lambda v: v.reshape(1, -1)
    return _tc_mlp(e1, e2, W1, r(b1), r(s1), r(t1v), W2, r(b2), r(s2), r(t2v))
